# baseline (speedup 1.0000x reference)
import functools

import jax
import jax.numpy as jnp
from jax import lax
from jax.experimental import pallas as pl
from jax.experimental.pallas import tpu as pltpu
from jax.experimental.pallas import tpu_sc as plsc

NC = 2
NS = 16
NW = NC * NS

V = 256
D = 64
BATCH = 1024
SEQ = 200
K = 4
Q = 8
NPB = BATCH // Q
M = Q * K
SB = 8
NT = SEQ // SB
BW = 16

_mesh = plsc.VectorSubcoreMesh(core_axis_name="c", subcore_axis_name="s")


@functools.partial(
    pl.kernel,
    mesh=_mesh,
    out_type=jax.ShapeDtypeStruct((BATCH, SEQ, K * D), jnp.float32),
    scratch_types=[
        pltpu.VMEM((V, D), jnp.float32),
        pltpu.VMEM((SB, K, NPB), jnp.int32),
        pltpu.VMEM((SB, K, NPB), jnp.int32),
        pltpu.VMEM((BW, SB, K * D), jnp.float32),
        pltpu.VMEM((BW, SB, K * D), jnp.float32),
        pltpu.SemaphoreType.DMA,
        pltpu.SemaphoreType.DMA,
        pltpu.SemaphoreType.DMA,
    ],
)
def _gather_kernel(idx_hbm, w_hbm, out_hbm, table_v, idx0, idx1,
                   rows0, rows1, sem0, sem1, semI):
    wid = lax.axis_index("s") * NC + lax.axis_index("c")
    q = wid // 4
    n0 = (wid % 4) * 32

    pltpu.sync_copy(w_hbm, table_v)

    ibufs = (idx0, idx1)
    slabs = ((rows0, sem0), (rows1, sem1))

    def idx_src(t):
        return idx_hbm.at[pl.ds(SB * t, SB), pl.ds(K * q, K), :]

    pltpu.async_copy(idx_src(0), idx0, semI)

    def do_t(t, idx_b, idx_n):
        pltpu.make_async_copy(idx_src(t), idx_b, semI).wait()

        @pl.when(t < NT - 1)
        def _():
            pltpu.async_copy(idx_src(t + 1), idx_n, semI)

        for h in range(2):
            rows_b, sem_b = slabs[h]
            b0 = NPB * q + n0 + BW * h
            out_sl = out_hbm.at[pl.ds(b0, BW), pl.ds(SB * t, SB), :]

            @pl.when(t >= 1)
            def _():
                pltpu.make_async_copy(rows_b, out_sl, sem_b).wait()

            @plsc.parallel_loop(0, SB * K, unroll=4)
            def pair_fn(g):
                sl = g // K
                k = g % K
                iv = idx_b[sl, k, pl.ds(n0 + BW * h, BW)]
                for j in range(BW):
                    row = iv[j]
                    for kk in range(D // 16):
                        rows_b[j, sl, pl.ds(k * D + kk * 16, 16)] = (
                            table_v[row, pl.ds(kk * 16, 16)])

            pltpu.async_copy(rows_b, out_sl, sem_b)

    def do_t2(t2, _):
        for p in range(2):
            do_t(t2 * 2 + p, ibufs[p], ibufs[1 - p])
        return 0

    lax.fori_loop(0, NT // 2, do_t2, 0)
    do_t(NT - 1, ibufs[(NT - 1) % 2], ibufs[NT % 2])

    for h in range(2):
        rows_b, sem_b = slabs[h]
        b0 = NPB * q + n0 + BW * h
        out_sl = out_hbm.at[pl.ds(b0, BW), pl.ds(SB * (NT - 1), SB), :]
        pltpu.make_async_copy(rows_b, out_sl, sem_b).wait()


def kernel(byte_indices, W):
    idx = byte_indices.astype(jnp.int32)
    idx3 = idx.reshape(Q, NPB, SEQ, K).transpose(2, 0, 3, 1).reshape(SEQ, M, NPB)
    return _gather_kernel(idx3, W.astype(jnp.float32))


# speedup vs baseline: 1.0857x; 1.0857x over previous
import functools

import jax
import jax.numpy as jnp
from jax import lax
from jax.experimental import pallas as pl
from jax.experimental.pallas import tpu as pltpu
from jax.experimental.pallas import tpu_sc as plsc

NC = 2
NS = 16
NW = NC * NS

V = 256
D = 64
BATCH = 1024
SEQ = 200
K = 4
Q = 8
NPB = BATCH // Q
M = Q * K
SB = 8
NT = SEQ // SB
BW = 16

_mesh = plsc.VectorSubcoreMesh(core_axis_name="c", subcore_axis_name="s")


@functools.partial(
    pl.kernel,
    mesh=_mesh,
    out_type=jax.ShapeDtypeStruct((BATCH, SEQ, K * D), jnp.float32),
    scratch_types=[
        pltpu.VMEM((V, D), jnp.float32),
        pltpu.VMEM((SB, K, NPB), jnp.int32),
        pltpu.VMEM((SB, K, NPB), jnp.int32),
        pltpu.VMEM((BW, SB, K * D), jnp.float32),
        pltpu.VMEM((BW, SB, K * D), jnp.float32),
        pltpu.SemaphoreType.DMA,
        pltpu.SemaphoreType.DMA,
        pltpu.SemaphoreType.DMA,
    ],
)
def _gather_kernel(idx_hbm, w_hbm, out_hbm, table_v, idx0, idx1,
                   rows0, rows1, sem0, sem1, semI):
    wid = lax.axis_index("s") * NC + lax.axis_index("c")
    q = wid // 4
    n0 = (wid % 4) * 32

    pltpu.sync_copy(w_hbm, table_v)

    ibufs = (idx0, idx1)
    slabs = ((rows0, sem0), (rows1, sem1))

    def idx_src(t):
        return idx_hbm.at[pl.ds(SB * t, SB), pl.ds(K * q, K), :]

    pltpu.async_copy(idx_src(0), idx0, semI)

    def do_t(t, idx_b, idx_n):
        pltpu.make_async_copy(idx_src(t), idx_b, semI).wait()

        @pl.when(t < NT - 1)
        def _():
            pltpu.async_copy(idx_src(t + 1), idx_n, semI)

        for h in range(2):
            rows_b, sem_b = slabs[h]
            b0 = NPB * q + n0 + BW * h
            out_sl = out_hbm.at[pl.ds(b0, BW), pl.ds(SB * t, SB), :]

            @pl.when(t >= 1)
            def _():
                pltpu.make_async_copy(rows_b, out_sl, sem_b).wait()

            @plsc.parallel_loop(0, SB * K, unroll=2)
            def pair_fn(g):
                sl = g // K
                k = g % K
                iv = idx_b[sl, k, pl.ds(n0 + BW * h, BW)]
                for j in range(BW):
                    row = iv[j]
                    for kk in range(D // 16):
                        rows_b[j, sl, pl.ds(k * D + kk * 16, 16)] = (
                            table_v[row, pl.ds(kk * 16, 16)])

            pltpu.async_copy(rows_b, out_sl, sem_b)

    def do_t2(t2, _):
        for p in range(2):
            do_t(t2 * 2 + p, ibufs[p], ibufs[1 - p])
        return 0

    lax.fori_loop(0, NT // 2, do_t2, 0)
    do_t(NT - 1, ibufs[(NT - 1) % 2], ibufs[NT % 2])

    for h in range(2):
        rows_b, sem_b = slabs[h]
        b0 = NPB * q + n0 + BW * h
        out_sl = out_hbm.at[pl.ds(b0, BW), pl.ds(SB * (NT - 1), SB), :]
        pltpu.make_async_copy(rows_b, out_sl, sem_b).wait()


def kernel(byte_indices, W):
    idx = byte_indices.astype(jnp.int32)
    idx3 = idx.reshape(Q, NPB, SEQ, K).transpose(2, 0, 3, 1).reshape(SEQ, M, NPB)
    return _gather_kernel(idx3, W.astype(jnp.float32))
